# baseline (speedup 1.0000x reference)
import jax
import jax.numpy as jnp
from jax.experimental import pallas as pl
from jax.experimental.pallas import tpu as pltpu

_PARENTS = (-1, 0, 1, 2, 0, 4, 5, 0, 7, 8, 9, 8, 11, 12, 8, 14, 15)
_N = 17
_W2 = tuple(1 if c % 2 == 1 else -1 for c in range(_N))

_BB = 128


def _body(fa_ref, attn_ref, pts_ref, vec_ref, out_ref):
    bb = attn_ref.shape[0]
    scores = jnp.sum(attn_ref[...], axis=(1, 3))
    node_i = jax.lax.broadcasted_iota(jnp.int32, (bb, _N), 1)

    m1 = jnp.max(scores, axis=1, keepdims=True)
    idx0 = jnp.min(jnp.where(scores == m1, node_i, _N), axis=1, keepdims=True)
    masked = jnp.where(node_i == idx0, -jnp.inf, scores)
    m2 = jnp.max(masked, axis=1, keepdims=True)
    idx1 = jnp.min(jnp.where(masked == m2, node_i, _N), axis=1, keepdims=True)

    def cls_w(idx):
        return (
            (idx == 0).astype(jnp.float32),
            ((idx >= 1) & (idx <= 3)).astype(jnp.float32),
            ((idx >= 4) & (idx <= 6)).astype(jnp.float32),
            (idx >= 7).astype(jnp.float32),
        )

    a = cls_w(idx0)
    b = cls_w(idx1)
    w = [a[k] + b[k] for k in range(4)]

    u0 = pts_ref[:, 0, :]
    u1 = pts_ref[:, 1, :] + vec_ref[:, 0, :]
    u4 = pts_ref[:, 4, :] - vec_ref[:, 3, :]
    u7 = pts_ref[:, 7, :] + vec_ref[:, 6, :]
    upd = 0.5 * (w[0] * u0 + w[1] * u1 + w[2] * u4 + w[3] * u7)

    fa = fa_ref[0]
    node_val = [None] * _N
    node_val[0] = upd
    out_ref[:, 0, :] = pts_ref[:, 0, :] + fa * upd
    for c in range(1, _N):
        v = node_val[_PARENTS[c]] - float(_W2[c]) * vec_ref[:, c - 1, :]
        node_val[c] = v
        out_ref[:, c, :] = pts_ref[:, c, :] + fa * v


@jax.jit
def kernel(points, vectors, attntion_scors, Fa):
    bsz = points.shape[0]
    grid = (bsz // _BB,)
    return pl.pallas_call(
        _body,
        grid=grid,
        in_specs=[
            pl.BlockSpec(memory_space=pltpu.SMEM),
            pl.BlockSpec((_BB, 8, _N, _N), lambda i: (i, 0, 0, 0)),
            pl.BlockSpec((_BB, _N, 256), lambda i: (i, 0, 0)),
            pl.BlockSpec((_BB, 16, 256), lambda i: (i, 0, 0)),
        ],
        out_specs=pl.BlockSpec((_BB, _N, 256), lambda i: (i, 0, 0)),
        out_shape=jax.ShapeDtypeStruct((bsz, _N, 256), points.dtype),
    )(Fa, attntion_scors, points, vectors)


# speedup vs baseline: 1.2261x; 1.2261x over previous
import numpy as np

import jax
import jax.numpy as jnp
from jax.experimental import pallas as pl
from jax.experimental.pallas import tpu as pltpu

_PARENTS = (-1, 0, 1, 2, 0, 4, 5, 0, 7, 8, 9, 8, 11, 12, 8, 14, 15)
_N = 17
_H = 8
_Z = 256
_W2 = tuple(1 if c % 2 == 1 else -1 for c in range(_N))

_BB = 256

_M_np = np.zeros((_H * _N * _N, _N), dtype=np.float32)
for _h in range(_H):
    for _n in range(_N):
        _M_np[_h * _N * _N + _n * _N : _h * _N * _N + (_n + 1) * _N, _n] = 1.0


def _body(fa_ref, m_ref, attn_ref, pts_ref, vec_ref, out_ref):
    bb = attn_ref.shape[0]
    scores = jnp.dot(attn_ref[...], m_ref[...], preferred_element_type=jnp.float32)
    node_i = jax.lax.broadcasted_iota(jnp.int32, (bb, _N), 1)

    m1 = jnp.max(scores, axis=1, keepdims=True)
    idx0 = jnp.min(jnp.where(scores == m1, node_i, _N), axis=1, keepdims=True)
    masked = jnp.where(node_i == idx0, -jnp.inf, scores)
    m2 = jnp.max(masked, axis=1, keepdims=True)
    idx1 = jnp.min(jnp.where(masked == m2, node_i, _N), axis=1, keepdims=True)

    def cls_w(idx):
        return (
            (idx == 0).astype(jnp.float32),
            ((idx >= 1) & (idx <= 3)).astype(jnp.float32),
            ((idx >= 4) & (idx <= 6)).astype(jnp.float32),
            (idx >= 7).astype(jnp.float32),
        )

    def pcol(c):
        return pts_ref[:, c * _Z : (c + 1) * _Z]

    def vcol(c):
        return vec_ref[:, c * _Z : (c + 1) * _Z]

    a = cls_w(idx0)
    b = cls_w(idx1)
    w = [a[k] + b[k] for k in range(4)]

    u0 = pcol(0)
    u1 = pcol(1) + vcol(0)
    u4 = pcol(4) - vcol(3)
    u7 = pcol(7) + vcol(6)
    upd = 0.5 * (w[0] * u0 + w[1] * u1 + w[2] * u4 + w[3] * u7)

    fa = fa_ref[0]
    node_val = [None] * _N
    node_val[0] = upd
    out_ref[:, 0:_Z] = pcol(0) + fa * upd
    for c in range(1, _N):
        v = node_val[_PARENTS[c]] - float(_W2[c]) * vcol(c - 1)
        node_val[c] = v
        out_ref[:, c * _Z : (c + 1) * _Z] = pcol(c) + fa * v


@jax.jit
def kernel(points, vectors, attntion_scors, Fa):
    bsz = points.shape[0]
    attn_flat = attntion_scors.reshape(bsz, _H * _N * _N)
    pts_flat = points.reshape(bsz, _N * _Z)
    vec_flat = vectors.reshape(bsz, 16 * _Z)
    m_const = jnp.asarray(_M_np)
    grid = (bsz // _BB,)
    out = pl.pallas_call(
        _body,
        grid=grid,
        in_specs=[
            pl.BlockSpec(memory_space=pltpu.SMEM),
            pl.BlockSpec((_H * _N * _N, _N), lambda i: (0, 0)),
            pl.BlockSpec((_BB, _H * _N * _N), lambda i: (i, 0)),
            pl.BlockSpec((_BB, _N * _Z), lambda i: (i, 0)),
            pl.BlockSpec((_BB, 16 * _Z), lambda i: (i, 0)),
        ],
        out_specs=pl.BlockSpec((_BB, _N * _Z), lambda i: (i, 0)),
        out_shape=jax.ShapeDtypeStruct((bsz, _N * _Z), points.dtype),
    )(Fa, m_const, attn_flat, pts_flat, vec_flat)
    return out.reshape(bsz, _N, _Z)
